# Initial kernel scaffold; baseline (speedup 1.0000x reference)
import jax
import jax.numpy as jnp
from jax.experimental import pallas as pl
from jax.experimental.pallas import tpu as pltpu
from jax.experimental.pallas import tpu_sc as plsc


_WINDOW = 128


def _gather_rows(table, flat_idx):
    num_indices = flat_idx.shape[0]
    emb = table.shape[1]
    idx2d = flat_idx.reshape(1, num_indices)
    mesh = plsc.VectorSubcoreMesh(core_axis_name="core",
                                  subcore_axis_name="subcore")

    @pl.kernel(out_type=jax.ShapeDtypeStruct((num_indices, emb), table.dtype),
               mesh=mesh)
    def kern(x_hbm, i_hbm, o_hbm):
        def body(i_vmem, o_vmem):
            pltpu.sync_copy(x_hbm.at[i_vmem.at[0]], o_vmem)

        pltpu.emit_pipeline(
            body,
            grid=(num_indices // _WINDOW,),
            in_specs=[pl.BlockSpec((1, _WINDOW), index_map=lambda i: (0, i))],
            out_specs=[pl.BlockSpec((_WINDOW, emb), index_map=lambda i: (i, 0))],
            core_axis_name=("core", "subcore"),
            dimension_semantics=(pltpu.PARALLEL,),
        )(i_hbm, o_hbm)

    return kern(table, idx2d)


def kernel(inputs, embedding_weights):
    flat = jnp.reshape(inputs, (-1,)).astype(jnp.int32)
    vectors = _gather_rows(embedding_weights, flat)
    return jnp.reshape(vectors, inputs.shape + (embedding_weights.shape[1],))


# speedup vs baseline: 1.4764x; 1.4764x over previous
import jax
import jax.numpy as jnp
from jax.experimental import pallas as pl
from jax.experimental.pallas import tpu as pltpu
from jax.experimental.pallas import tpu_sc as plsc


_WINDOW = 128


def _gather_rows(table, flat_idx):
    num_indices = flat_idx.shape[0]
    emb = table.shape[1]
    idx2d = flat_idx.reshape(1, num_indices)
    mesh = plsc.VectorSubcoreMesh(core_axis_name="core",
                                  subcore_axis_name="subcore")

    @pl.kernel(out_type=jax.ShapeDtypeStruct((num_indices, emb), table.dtype),
               mesh=mesh,
               compiler_params=pltpu.CompilerParams(use_tc_tiling_on_sc=False))
    def kern(x_hbm, i_hbm, o_hbm):
        def body(i_vmem, o_vmem):
            pltpu.sync_copy(x_hbm.at[i_vmem.at[0]], o_vmem)

        pltpu.emit_pipeline(
            body,
            grid=(num_indices // _WINDOW,),
            in_specs=[pl.BlockSpec((1, _WINDOW), index_map=lambda i: (0, i))],
            out_specs=[pl.BlockSpec((_WINDOW, emb), index_map=lambda i: (i, 0))],
            core_axis_name=("core", "subcore"),
            dimension_semantics=(pltpu.PARALLEL,),
        )(i_hbm, o_hbm)

    return kern(table, idx2d)


def kernel(inputs, embedding_weights):
    flat = jnp.reshape(inputs, (-1,)).astype(jnp.int32)
    vectors = _gather_rows(embedding_weights, flat)
    return jnp.reshape(vectors, inputs.shape + (embedding_weights.shape[1],))


# speedup vs baseline: 1.5604x; 1.0569x over previous
import jax
import jax.numpy as jnp
from jax.experimental import pallas as pl
from jax.experimental.pallas import tpu as pltpu
from jax.experimental.pallas import tpu_sc as plsc


_WINDOW = 512


def _gather_rows(table, flat_idx):
    num_indices = flat_idx.shape[0]
    emb = table.shape[1]
    idx2d = flat_idx.reshape(1, num_indices)
    mesh = plsc.VectorSubcoreMesh(core_axis_name="core",
                                  subcore_axis_name="subcore")

    @pl.kernel(out_type=jax.ShapeDtypeStruct((num_indices, emb), table.dtype),
               mesh=mesh,
               compiler_params=pltpu.CompilerParams(use_tc_tiling_on_sc=False))
    def kern(x_hbm, i_hbm, o_hbm):
        def body(i_vmem, o_vmem):
            pltpu.sync_copy(x_hbm.at[i_vmem.at[0]], o_vmem)

        pltpu.emit_pipeline(
            body,
            grid=(num_indices // _WINDOW,),
            in_specs=[pl.BlockSpec((1, _WINDOW), index_map=lambda i: (0, i))],
            out_specs=[pl.BlockSpec((_WINDOW, emb), index_map=lambda i: (i, 0))],
            core_axis_name=("core", "subcore"),
            dimension_semantics=(pltpu.PARALLEL,),
        )(i_hbm, o_hbm)

    return kern(table, idx2d)


def kernel(inputs, embedding_weights):
    flat = jnp.reshape(inputs, (-1,)).astype(jnp.int32)
    vectors = _gather_rows(embedding_weights, flat)
    return jnp.reshape(vectors, inputs.shape + (embedding_weights.shape[1],))


# speedup vs baseline: 1.5693x; 1.0057x over previous
import jax
import jax.numpy as jnp
from jax.experimental import pallas as pl
from jax.experimental.pallas import tpu as pltpu
from jax.experimental.pallas import tpu_sc as plsc


_WINDOW = 1024


def _gather_rows(table, idx_fm, batch, fields):
    emb = table.shape[1]
    mesh = plsc.VectorSubcoreMesh(core_axis_name="core",
                                  subcore_axis_name="subcore")

    @pl.kernel(out_type=jax.ShapeDtypeStruct((batch, fields, emb), table.dtype),
               mesh=mesh,
               compiler_params=pltpu.CompilerParams(use_tc_tiling_on_sc=False))
    def kern(x_hbm, i_hbm, o_hbm):
        def body(i_vmem, o_vmem):
            pltpu.sync_copy(x_hbm.at[i_vmem.at[0]], o_vmem.at[:, 0])

        pltpu.emit_pipeline(
            body,
            grid=(fields, batch // _WINDOW),
            in_specs=[pl.BlockSpec((1, _WINDOW), index_map=lambda f, j: (f, j))],
            out_specs=[pl.BlockSpec((_WINDOW, 1, emb),
                                    index_map=lambda f, j: (j, f, 0))],
            core_axis_name=("core", "subcore"),
            dimension_semantics=(pltpu.PARALLEL, pltpu.PARALLEL),
        )(i_hbm, o_hbm)

    return kern(table, idx_fm)


def kernel(inputs, embedding_weights):
    batch, fields = inputs.shape
    idx_fm = jnp.transpose(inputs).astype(jnp.int32)
    return _gather_rows(embedding_weights, idx_fm, batch, fields)
